# baseline (speedup 1.0000x reference)
import jax
import jax.numpy as jnp
from jax.experimental import pallas as pl
from jax.experimental.pallas import tpu as pltpu

_DILATION_RATES = (1, 2)
_SEQ = 12
_N = 207
_D = 64
_BATCH = 16
_TP = 16
_R = _BATCH * _TP
_SPECIAL_NODE = (_N - 1) // 2

_DNT = (((1,), (1,)), ((), ()))


def _stgcn_kernel(x0_ref, x1_ref, x2_ref, x3_ref,
                  x4_ref, x5_ref, x6_ref, x7_ref,
                  wdyn_hbm, wsta_hbm, convw_hbm, gw_hbm, wa_hbm,
                  bd_hbm, bs_hbm, cb_hbm, gb_hbm, ba_hbm, v_hbm, um_hbm,
                  out_ref,
                  m_scr, fin_scr, wdyn, wsta, cw, gw, wa, brow, gbuf, babuf,
                  vbuf, umbuf, sems):
    s = pl.program_id(0)

    def weight_copies():
        return [
            pltpu.make_async_copy(wdyn_hbm, wdyn, sems.at[0]),
            pltpu.make_async_copy(wsta_hbm, wsta, sems.at[1]),
            pltpu.make_async_copy(convw_hbm, cw, sems.at[2]),
            pltpu.make_async_copy(gw_hbm, gw, sems.at[3]),
            pltpu.make_async_copy(wa_hbm, wa, sems.at[4]),
            pltpu.make_async_copy(bd_hbm, brow.at[0:2], sems.at[5]),
            pltpu.make_async_copy(bs_hbm, brow.at[2:4], sems.at[6]),
            pltpu.make_async_copy(cb_hbm, brow.at[4:6], sems.at[7]),
            pltpu.make_async_copy(gb_hbm, gbuf, sems.at[8]),
            pltpu.make_async_copy(ba_hbm, babuf, sems.at[9]),
            pltpu.make_async_copy(v_hbm, vbuf, sems.at[10]),
            pltpu.make_async_copy(um_hbm, umbuf, sems.at[11]),
        ]

    @pl.when(s == 0)
    def _start_weights():
        for c in weight_copies():
            c.start()

    @pl.when(s < 2)
    def _reduce():
        base = s * 8
        inv_n = 1.0 / _N
        m_scr[base + 0, :_SEQ] = jnp.sum(x0_ref[0], axis=2) * inv_n
        m_scr[base + 1, :_SEQ] = jnp.sum(x1_ref[0], axis=2) * inv_n
        m_scr[base + 2, :_SEQ] = jnp.sum(x2_ref[0], axis=2) * inv_n
        m_scr[base + 3, :_SEQ] = jnp.sum(x3_ref[0], axis=2) * inv_n
        m_scr[base + 4, :_SEQ] = jnp.sum(x4_ref[0], axis=2) * inv_n
        m_scr[base + 5, :_SEQ] = jnp.sum(x5_ref[0], axis=2) * inv_n
        m_scr[base + 6, :_SEQ] = jnp.sum(x6_ref[0], axis=2) * inv_n
        m_scr[base + 7, :_SEQ] = jnp.sum(x7_ref[0], axis=2) * inv_n

    @pl.when(s == 1)
    def _finalize():
        for c in weight_copies():
            c.wait()

        def dot_t(x, w):
            return jax.lax.dot_general(x, w, _DNT,
                                       preferred_element_type=jnp.float32)

        m = m_scr[...].reshape(_R, _D)
        um_flag = (umbuf[0, 0] != 0).astype(jnp.float32)
        tmod = jax.lax.broadcasted_iota(jnp.int32, (_R, _D), 0) & (_TP - 1)
        res = []
        for l, dil in enumerate(_DILATION_RATES):
            pr = jax.lax.broadcasted_iota(jnp.int32, (2 * _D, _D), 0)
            pc = jax.lax.broadcasted_iota(jnp.int32, (2 * _D, _D), 1)
            w2 = cw[l]
            wk0 = jnp.dot(w2, (pr == 2 * pc).astype(jnp.float32),
                          preferred_element_type=jnp.float32)
            wk1 = jnp.dot(w2, (pr == 2 * pc + 1).astype(jnp.float32),
                          preferred_element_type=jnp.float32)
            g_dyn = dot_t(m, wdyn[l]) + brow[l:l + 1, :]
            g_sta = dot_t(m, wsta[l]) + brow[2 + l:3 + l, :]
            pre = (dot_t(g_sta, gw[:, :_D]) + dot_t(g_dyn, gw[:, _D:])
                   + gbuf[...].reshape(1, _D))
            gated = jax.nn.sigmoid(pre)
            g = g_dyn + um_flag * (gated - g_dyn)
            gshift = jnp.where(tmod < dil, 0.0,
                               jnp.concatenate(
                                   [jnp.zeros((dil, _D), dtype=jnp.float32),
                                    g[:_R - dil]], axis=0))
            y = jax.nn.relu(
                dot_t(gshift, wk0)
                + dot_t(g, wk1)
                + brow[4 + l:5 + l, :])
            res.append(y.reshape(_BATCH, _TP, _D)[:, _SEQ - 1, :])
            m = m + y

        r1, r2 = res
        ba_row = babuf[...].reshape(1, _D)
        t1 = jnp.tanh(jnp.dot(r1, wa[...], preferred_element_type=jnp.float32)
                      + ba_row)
        t2 = jnp.tanh(jnp.dot(r2, wa[...], preferred_element_type=jnp.float32)
                      + ba_row)
        s1 = jnp.dot(t1, vbuf[...], preferred_element_type=jnp.float32)
        s2 = jnp.dot(t2, vbuf[...], preferred_element_type=jnp.float32)
        mx = jnp.maximum(s1, s2)
        e1 = jnp.exp(s1 - mx)
        e2 = jnp.exp(s2 - mx)
        a0 = e1 / (e1 + e2)
        fin_scr[0] = 0.5 * (r1 + r2)
        fin_scr[1] = a0 * r1 + (1.0 - a0) * r2

    @pl.when(s >= 2)
    def _write():
        base = 4 * s - 8
        mean4 = fin_scr[0, pl.ds(base, 4), :]
        spec4 = fin_scr[1, pl.ds(base, 4), :]
        rows = jax.lax.broadcasted_iota(jnp.int32, (1, _N, _D), 1)
        out_ref[...] = jnp.where(rows == _SPECIAL_NODE,
                                 spec4[:, None, :], mean4[:, None, :])


def kernel(node_embeddings, B, static_MTE_matrix, W_dyn, b_dyn, W_sta, b_sta,
           conv_w, conv_b, gate_W, gate_b, Wa, ba, v, use_MTE):
    batch, seq, d, N = node_embeddings.shape
    um = jnp.asarray(use_MTE, jnp.int32).reshape(1, 1)
    cwt = conv_w.reshape(conv_w.shape[0], d, 2 * d)

    def stream(k):
        return pl.BlockSpec((1, seq, d, N),
                            lambda s, k=k: (jnp.minimum(s, 1) * 8 + k, 0, 0, 0))

    hbm = pl.BlockSpec(memory_space=pl.ANY)

    out = pl.pallas_call(
        _stgcn_kernel,
        grid=(6,),
        in_specs=[stream(k) for k in range(8)] + [hbm] * 12,
        out_specs=pl.BlockSpec((4, N, d), lambda s: (jnp.maximum(s - 2, 0), 0, 0)),
        out_shape=jax.ShapeDtypeStruct((batch, N, d), jnp.float32),
        scratch_shapes=[
            pltpu.VMEM((_BATCH, _TP, _D), jnp.float32),
            pltpu.VMEM((2, _BATCH, _D), jnp.float32),
            pltpu.VMEM((2, _D, _D), jnp.float32),
            pltpu.VMEM((2, _D, _D), jnp.float32),
            pltpu.VMEM((2, _D, 2 * _D), jnp.float32),
            pltpu.VMEM((_D, 2 * _D), jnp.float32),
            pltpu.VMEM((_D, _D), jnp.float32),
            pltpu.VMEM((8, _D), jnp.float32),
            pltpu.VMEM((_D,), jnp.float32),
            pltpu.VMEM((_D,), jnp.float32),
            pltpu.VMEM((_D, 1), jnp.float32),
            pltpu.VMEM((1, 1), jnp.int32),
            pltpu.SemaphoreType.DMA((12,)),
        ],
    )(node_embeddings, node_embeddings, node_embeddings, node_embeddings,
      node_embeddings, node_embeddings, node_embeddings, node_embeddings,
      W_dyn, W_sta, cwt, gate_W, Wa, b_dyn, b_sta, conv_b, gate_b, ba, v, um)
    return out


# speedup vs baseline: 1.0342x; 1.0342x over previous
import jax
import jax.numpy as jnp
from jax.experimental import pallas as pl
from jax.experimental.pallas import tpu as pltpu

_DILATION_RATES = (1, 2)
_SEQ = 12
_N = 207
_D = 64
_BATCH = 16
_TP = 16
_R = _BATCH * _TP
_SPECIAL_NODE = (_N - 1) // 2

_DNT = (((1,), (1,)), ((), ()))


def _stgcn_kernel(x0_ref, x1_ref, x2_ref, x3_ref,
                  x4_ref, x5_ref, x6_ref, x7_ref,
                  wdyn_hbm, wsta_hbm, convw_hbm, gw_hbm, wa_hbm,
                  bd_hbm, bs_hbm, cb_hbm, gb_hbm, ba_hbm, v_hbm, um_hbm,
                  out_ref,
                  m_scr, fin_scr, wdyn, wsta, cw, gw, wa, brow, gbuf, babuf,
                  vbuf, umbuf, sems):
    s = pl.program_id(0)

    def weight_copies():
        return [
            pltpu.make_async_copy(wdyn_hbm, wdyn, sems.at[0]),
            pltpu.make_async_copy(wsta_hbm, wsta, sems.at[1]),
            pltpu.make_async_copy(convw_hbm, cw, sems.at[2]),
            pltpu.make_async_copy(gw_hbm, gw, sems.at[3]),
            pltpu.make_async_copy(wa_hbm, wa, sems.at[4]),
            pltpu.make_async_copy(bd_hbm, brow.at[0:2], sems.at[5]),
            pltpu.make_async_copy(bs_hbm, brow.at[2:4], sems.at[6]),
            pltpu.make_async_copy(cb_hbm, brow.at[4:6], sems.at[7]),
            pltpu.make_async_copy(gb_hbm, gbuf, sems.at[8]),
            pltpu.make_async_copy(ba_hbm, babuf, sems.at[9]),
            pltpu.make_async_copy(v_hbm, vbuf, sems.at[10]),
            pltpu.make_async_copy(um_hbm, umbuf, sems.at[11]),
        ]

    @pl.when(s == 0)
    def _start_weights():
        for c in weight_copies():
            c.start()

    @pl.when(s < 2)
    def _reduce():
        base = s * 8
        inv_n = 1.0 / _N
        m_scr[base + 0, :_SEQ] = jnp.sum(x0_ref[0], axis=2) * inv_n
        m_scr[base + 1, :_SEQ] = jnp.sum(x1_ref[0], axis=2) * inv_n
        m_scr[base + 2, :_SEQ] = jnp.sum(x2_ref[0], axis=2) * inv_n
        m_scr[base + 3, :_SEQ] = jnp.sum(x3_ref[0], axis=2) * inv_n
        m_scr[base + 4, :_SEQ] = jnp.sum(x4_ref[0], axis=2) * inv_n
        m_scr[base + 5, :_SEQ] = jnp.sum(x5_ref[0], axis=2) * inv_n
        m_scr[base + 6, :_SEQ] = jnp.sum(x6_ref[0], axis=2) * inv_n
        m_scr[base + 7, :_SEQ] = jnp.sum(x7_ref[0], axis=2) * inv_n

    @pl.when(s == 1)
    def _finalize():
        for c in weight_copies():
            c.wait()

        def dot_t(x, w):
            return jax.lax.dot_general(x, w, _DNT,
                                       preferred_element_type=jnp.float32)

        m = m_scr[...].reshape(_R, _D)
        um_flag = (umbuf[0, 0] != 0).astype(jnp.float32)
        tmod = jax.lax.broadcasted_iota(jnp.int32, (_R, _D), 0) & (_TP - 1)
        res = []
        for l, dil in enumerate(_DILATION_RATES):
            pr = jax.lax.broadcasted_iota(jnp.int32, (2 * _D, _D), 0)
            pc = jax.lax.broadcasted_iota(jnp.int32, (2 * _D, _D), 1)
            w2 = cw[l]
            wk0 = jnp.dot(w2, (pr == 2 * pc).astype(jnp.float32),
                          preferred_element_type=jnp.float32)
            wk1 = jnp.dot(w2, (pr == 2 * pc + 1).astype(jnp.float32),
                          preferred_element_type=jnp.float32)
            g_dyn = dot_t(m, wdyn[l]) + brow[l:l + 1, :]
            g_sta = dot_t(m, wsta[l]) + brow[2 + l:3 + l, :]
            pre = (dot_t(g_sta, gw[:, :_D]) + dot_t(g_dyn, gw[:, _D:])
                   + gbuf[...].reshape(1, _D))
            gated = jax.nn.sigmoid(pre)
            g = g_dyn + um_flag * (gated - g_dyn)
            gshift = jnp.where(tmod < dil, 0.0,
                               jnp.concatenate(
                                   [jnp.zeros((dil, _D), dtype=jnp.float32),
                                    g[:_R - dil]], axis=0))
            y = jax.nn.relu(
                dot_t(gshift, wk0)
                + dot_t(g, wk1)
                + brow[4 + l:5 + l, :])
            res.append(y.reshape(_BATCH, _TP, _D)[:, _SEQ - 1, :])
            m = m + y

        r1, r2 = res
        ba_row = babuf[...].reshape(1, _D)
        t1 = jnp.tanh(jnp.dot(r1, wa[...], preferred_element_type=jnp.float32)
                      + ba_row)
        t2 = jnp.tanh(jnp.dot(r2, wa[...], preferred_element_type=jnp.float32)
                      + ba_row)
        s1 = jnp.dot(t1, vbuf[...], preferred_element_type=jnp.float32)
        s2 = jnp.dot(t2, vbuf[...], preferred_element_type=jnp.float32)
        mx = jnp.maximum(s1, s2)
        e1 = jnp.exp(s1 - mx)
        e2 = jnp.exp(s2 - mx)
        a0 = e1 / (e1 + e2)
        fin_scr[0] = 0.5 * (r1 + r2)
        fin_scr[1] = a0 * r1 + (1.0 - a0) * r2

    @pl.when(s == 2)
    def _write():
        rows = jax.lax.broadcasted_iota(jnp.int32, (1, _N, _D), 1)
        spec = fin_scr[1]
        mean = fin_scr[0]
        out_ref[...] = jnp.where(rows == _SPECIAL_NODE,
                                 spec[:, None, :], mean[:, None, :])


def kernel(node_embeddings, B, static_MTE_matrix, W_dyn, b_dyn, W_sta, b_sta,
           conv_w, conv_b, gate_W, gate_b, Wa, ba, v, use_MTE):
    batch, seq, d, N = node_embeddings.shape
    um = jnp.asarray(use_MTE, jnp.int32).reshape(1, 1)
    cwt = conv_w.reshape(conv_w.shape[0], d, 2 * d)

    def stream(k):
        return pl.BlockSpec((1, seq, d, N),
                            lambda s, k=k: (jnp.minimum(s, 1) * 8 + k, 0, 0, 0))

    hbm = pl.BlockSpec(memory_space=pl.ANY)

    out = pl.pallas_call(
        _stgcn_kernel,
        grid=(3,),
        in_specs=[stream(k) for k in range(8)] + [hbm] * 12,
        out_specs=pl.BlockSpec((batch, N, d), lambda s: (0, 0, 0)),
        out_shape=jax.ShapeDtypeStruct((batch, N, d), jnp.float32),
        scratch_shapes=[
            pltpu.VMEM((_BATCH, _TP, _D), jnp.float32),
            pltpu.VMEM((2, _BATCH, _D), jnp.float32),
            pltpu.VMEM((2, _D, _D), jnp.float32),
            pltpu.VMEM((2, _D, _D), jnp.float32),
            pltpu.VMEM((2, _D, 2 * _D), jnp.float32),
            pltpu.VMEM((_D, 2 * _D), jnp.float32),
            pltpu.VMEM((_D, _D), jnp.float32),
            pltpu.VMEM((8, _D), jnp.float32),
            pltpu.VMEM((_D,), jnp.float32),
            pltpu.VMEM((_D,), jnp.float32),
            pltpu.VMEM((_D, 1), jnp.float32),
            pltpu.VMEM((1, 1), jnp.int32),
            pltpu.SemaphoreType.DMA((12,)),
        ],
    )(node_embeddings, node_embeddings, node_embeddings, node_embeddings,
      node_embeddings, node_embeddings, node_embeddings, node_embeddings,
      W_dyn, W_sta, cwt, gate_W, Wa, b_dyn, b_sta, conv_b, gate_b, ba, v, um)
    return out
